# Initial kernel scaffold; baseline (speedup 1.0000x reference)
import jax
import jax.numpy as jnp
from jax.experimental import pallas as pl


def kernel(x, edge_index, adj_norm_sp, W_l, b_l, W_r, gamma, beta):
    raise NotImplementedError("write your pallas kernel here")


# speedup vs baseline: 11.1999x; 11.1999x over previous
import functools

import jax
import jax.numpy as jnp
from jax import lax
from jax.experimental import pallas as pl
from jax.experimental.pallas import tpu as pltpu
from jax.experimental.pallas import tpu_sc as plsc

_N = 10000
_D = 128
_H = 128
_E = 320000
_EPS_BN = 1e-5

_NC = 2
_NS = 16
_NW = _NC * _NS
_EPW = _E // _NW
_CK = 80
_NCHUNK = _EPW // _CK
_NPAD = 10240
_RPT = 640
_RPT_TAIL = _N - 15 * _RPT


def _sc_aggregate(x, src, dst, zrows):
    mesh = plsc.VectorSubcoreMesh(core_axis_name="c", subcore_axis_name="s")

    @functools.partial(
        pl.kernel,
        out_type=(jax.ShapeDtypeStruct((_NC, _N, _D), jnp.float32),
                  jax.ShapeDtypeStruct((_NW, _NPAD), jnp.float32)),
        mesh=mesh,
        compiler_params=pltpu.CompilerParams(needs_layout_passes=False),
        scratch_types=[
            pltpu.VMEM((_CK, _D), jnp.float32),
            pltpu.VMEM((_CK, _D), jnp.float32),
            pltpu.VMEM((_CK,), jnp.int32),
            pltpu.VMEM((_CK,), jnp.int32),
            pltpu.VMEM((_CK,), jnp.int32),
            pltpu.VMEM((_CK,), jnp.int32),
            pltpu.VMEM((_NPAD,), jnp.float32),
            pltpu.VMEM_SHARED((_N, _D), jnp.float32),
            pltpu.SemaphoreType.DMA,
            pltpu.SemaphoreType.DMA,
            pltpu.SemaphoreType.DMA,
            pltpu.SemaphoreType.DMA,
        ],
    )
    def k(x_hbm, src_hbm, dst_hbm, z_hbm, out_hbm, deg_hbm,
          rows0, rows1, srcc0, srcc1, dstc0, dstc1, deg_local, agg_sh,
          gsem0, gsem1, isem0, isem1):
        cid = lax.axis_index("c")
        sid = lax.axis_index("s")
        wid = sid * _NC + cid
        ebase = pl.multiple_of(wid * _EPW, 8)
        rbase = pl.multiple_of(sid * _RPT, 8)

        @pl.when(sid < _NS - 1)
        def _():
            pltpu.sync_copy(z_hbm, agg_sh.at[pl.ds(rbase, _RPT)])

        @pl.when(sid == _NS - 1)
        def _():
            pltpu.sync_copy(z_hbm.at[pl.ds(0, _RPT_TAIL)],
                            agg_sh.at[pl.ds(15 * _RPT, _RPT_TAIL)])

        zvec = jnp.zeros((16,), jnp.float32)

        def zero_body(j, carry):
            deg_local[pl.ds(j * 16, 16)] = zvec
            return carry

        lax.fori_loop(0, _NPAD // 16, zero_body, 0)
        plsc.subcore_barrier()

        rows = (rows0, rows1)
        srcc = (srcc0, srcc1)
        dstc = (dstc0, dstc1)
        gsems = (gsem0, gsem1)
        isems = (isem0, isem1)

        def start_idx(c, b):
            off = pl.multiple_of(ebase + c * _CK, 8)
            pltpu.async_copy(src_hbm.at[pl.ds(off, _CK)], srcc[b], isems[b])
            pltpu.async_copy(dst_hbm.at[pl.ds(off, _CK)], dstc[b], isems[b])

        def wait_idx(b):
            pltpu.make_async_copy(src_hbm.at[pl.ds(0, _CK)], srcc[b],
                                  isems[b]).wait()
            pltpu.make_async_copy(dst_hbm.at[pl.ds(0, _CK)], dstc[b],
                                  isems[b]).wait()

        def start_gather(b):
            pltpu.async_copy(x_hbm.at[srcc[b]], rows[b], gsems[b])

        def wait_gather(b):
            pltpu.make_async_copy(x_hbm.at[pl.ds(0, _CK)], rows[b],
                                  gsems[b]).wait()

        ones16 = jnp.ones((16,), jnp.float32)

        def hist(b):
            for j in range(_CK // 16):
                idx = dstc[b][pl.ds(j * 16, 16)]
                plsc.addupdate_scatter(deg_local, [idx], ones16)

        def scatter(b):
            pltpu.sync_copy(rows[b], agg_sh.at[dstc[b]], add=True)

        start_idx(0, 0)
        wait_idx(0)
        start_gather(0)
        start_idx(1, 1)

        def body(i, carry):
            a = 2 * i
            wait_idx(1)
            start_gather(1)
            hist(0)
            wait_gather(0)
            scatter(0)

            @pl.when(a + 2 < _NCHUNK)
            def _():
                start_idx(a + 2, 0)

            hist(1)

            @pl.when(a + 2 < _NCHUNK)
            def _():
                wait_idx(0)
                start_gather(0)

            wait_gather(1)
            scatter(1)

            @pl.when(a + 3 < _NCHUNK)
            def _():
                start_idx(a + 3, 1)

            return carry

        assert _NCHUNK % 2 == 1
        lax.fori_loop(0, _NCHUNK // 2, body, 0)
        hist(0)
        wait_gather(0)
        scatter(0)

        pltpu.sync_copy(deg_local, deg_hbm.at[wid])
        plsc.subcore_barrier()

        @pl.when(sid < _NS - 1)
        def _():
            pltpu.sync_copy(agg_sh.at[pl.ds(rbase, _RPT)],
                            out_hbm.at[cid, pl.ds(rbase, _RPT)])

        @pl.when(sid == _NS - 1)
        def _():
            pltpu.sync_copy(agg_sh.at[pl.ds(15 * _RPT, _RPT_TAIL)],
                            out_hbm.at[cid, pl.ds(15 * _RPT, _RPT_TAIL)])

    return k(x, src, dst, zrows)


def _tc_body(x_ref, p_ref, dp_ref, wl_ref, bl_ref, wr_ref, g_ref, b_ref,
             o_ref):
    agg = p_ref[0] + p_ref[1]
    deg = jnp.sum(dp_ref[...], axis=0)[:_N]
    agg = agg / jnp.maximum(deg, 1.0)[:, None]
    h = (jnp.dot(agg, wl_ref[...], preferred_element_type=jnp.float32)
         + jnp.dot(x_ref[...], wr_ref[...], preferred_element_type=jnp.float32)
         + bl_ref[...])
    nrm = jnp.sqrt(jnp.sum(h * h, axis=-1, keepdims=True))
    h = h / jnp.maximum(nrm, 1e-12)
    h = jnp.maximum(h, 0.0)
    mean = jnp.mean(h, axis=0, keepdims=True)
    var = jnp.mean((h - mean) ** 2, axis=0, keepdims=True)
    o_ref[...] = (h - mean) * (g_ref[...] * lax.rsqrt(var + _EPS_BN)) + b_ref[...]


def kernel(x, edge_index, adj_norm_sp, W_l, b_l, W_r, gamma, beta):
    del adj_norm_sp
    zrows = jnp.zeros((_RPT, _D), jnp.float32)
    ei = edge_index.astype(jnp.int32)
    parts, deg_parts = _sc_aggregate(x, ei[0], ei[1], zrows)
    return pl.pallas_call(
        _tc_body,
        out_shape=jax.ShapeDtypeStruct((_N, _H), jnp.float32),
    )(x, parts, deg_parts, W_l, b_l.reshape(1, _H), W_r,
      gamma.reshape(1, _H), beta.reshape(1, _H))
